# baseline (speedup 1.0000x reference)
import jax
import jax.numpy as jnp
from jax import lax
from jax.experimental import pallas as pl
from jax.experimental.pallas import tpu as pltpu
from jax.experimental.pallas import tpu_sc as plsc

VOCAB = 100000
D = 2048
B = 4
T = 2048
N = B * T

NC = 2
NS = 16
NW = NC * NS
T_PER_W = T // NW
RG = 16
RP = 8
NCHUNK = T_PER_W // RG
NSTEP = NCHUNK * B
NPOS = T_PER_W // RP
LANES = 16
DV = D // LANES


def _add_half(tok_v, half, pos_v):
    @pl.loop(0, RP)
    def row_loop(r):
        @pl.loop(0, DV, unroll=16)
        def col_loop(j):
            sl = pl.ds(j * LANES, LANES)
            plsc.addupdate(tok_v.at[half * RP + r, sl], pos_v[r, sl])


def _body(
    idx_hbm, tok_hbm, pos_hbm, out_hbm,
    idx_v, tok0, tok1, pos0, pos1, pos2,
    sg0, sg1, sw0, sw1, sp0, sp1, sp2, si,
):
    c = lax.axis_index("c")
    s = lax.axis_index("s")
    wid = s * NC + c
    t0 = wid * T_PER_W

    toks = (tok0, tok1)
    poss = (pos0, pos1, pos2)
    sgs = (sg0, sg1)
    sws = (sw0, sw1)
    sps = (sp0, sp1, sp2)

    for b in range(B):
        pltpu.async_copy(
            idx_hbm.at[pl.ds(b * T + t0, T_PER_W)],
            idx_v.at[pl.ds(b * T_PER_W, T_PER_W)],
            si,
        )
    for b in range(B):
        pltpu.make_async_copy(
            idx_hbm.at[pl.ds(0, T_PER_W)],
            idx_v.at[pl.ds(b * T_PER_W, T_PER_W)],
            si,
        ).wait()

    def fetch_tok(k):
        ci, b = k // B, k % B
        slot = k % 2
        pltpu.async_copy(
            tok_hbm.at[idx_v.at[pl.ds(b * T_PER_W + ci * RG, RG)]], toks[slot], sgs[slot]
        )

    def wait_tok(k):
        slot = k % 2
        pltpu.make_async_copy(tok_hbm.at[pl.ds(0, RG)], toks[slot], sgs[slot]).wait()

    def start_write(k):
        ci, b = k // B, k % B
        slot = k % 2
        pltpu.async_copy(
            toks[slot], out_hbm.at[pl.ds(b * T + t0 + ci * RG, RG)], sws[slot]
        )

    def wait_write(k):
        slot = k % 2
        pltpu.make_async_copy(toks[slot], out_hbm.at[pl.ds(0, RG)], sws[slot]).wait()

    def fetch_pos(cp):
        pltpu.async_copy(
            pos_hbm.at[pl.ds(t0 + cp * RP, RP)], poss[cp % 3], sps[cp % 3]
        )

    def wait_pos(cp):
        pltpu.make_async_copy(
            pos_hbm.at[pl.ds(0, RP)], poss[cp % 3], sps[cp % 3]
        ).wait()

    fetch_pos(0)
    fetch_pos(1)
    fetch_tok(0)

    for k in range(NSTEP):
        ci, b = k // B, k % B
        if k + 1 < NSTEP:
            if k >= 1:
                wait_write(k - 1)
            fetch_tok(k + 1)
        if b == 0:
            wait_pos(2 * ci)
            if 2 * ci + 2 < NPOS:
                fetch_pos(2 * ci + 2)
        wait_tok(k)
        _add_half(toks[k % 2], 0, poss[(2 * ci) % 3])
        if b == B - 1 and 2 * ci + 3 < NPOS:
            fetch_pos(2 * ci + 3)
        if b == 0:
            wait_pos(2 * ci + 1)
        _add_half(toks[k % 2], 1, poss[(2 * ci + 1) % 3])
        start_write(k)

    wait_write(NSTEP - 2)
    wait_write(NSTEP - 1)


@jax.jit
def _run(idx_flat, tok_table, pos_table):
    mesh = plsc.VectorSubcoreMesh(
        core_axis_name="c", subcore_axis_name="s", num_cores=NC, num_subcores=NS
    )
    f = pl.kernel(
        _body,
        out_type=jax.ShapeDtypeStruct((N, D), jnp.float32),
        mesh=mesh,
        scratch_types=[
            pltpu.VMEM((B * T_PER_W,), jnp.int32),
            pltpu.VMEM((RG, D), jnp.float32),
            pltpu.VMEM((RG, D), jnp.float32),
            pltpu.VMEM((RP, D), jnp.float32),
            pltpu.VMEM((RP, D), jnp.float32),
            pltpu.VMEM((RP, D), jnp.float32),
            pltpu.SemaphoreType.DMA,
            pltpu.SemaphoreType.DMA,
            pltpu.SemaphoreType.DMA,
            pltpu.SemaphoreType.DMA,
            pltpu.SemaphoreType.DMA,
            pltpu.SemaphoreType.DMA,
            pltpu.SemaphoreType.DMA,
            pltpu.SemaphoreType.DMA,
        ],
    )
    return f(idx_flat, tok_table, pos_table)


def kernel(idx, tok_table, pos_table):
    idx_flat = idx.reshape(N).astype(jnp.int32)
    out = _run(idx_flat, tok_table, pos_table)
    return out.reshape(B, T, D)


# speedup vs baseline: 1.1038x; 1.1038x over previous
import jax
import jax.numpy as jnp
from jax import lax
from jax.experimental import pallas as pl
from jax.experimental.pallas import tpu as pltpu
from jax.experimental.pallas import tpu_sc as plsc

VOCAB = 100000
D = 2048
B = 4
T = 2048
N = B * T

NC = 2
NS = 16
NW = NC * NS
T_PER_W = T // NW
R = 8
NCHUNK = T_PER_W // R
NSTEP = NCHUNK * B
NSLOT = 5
LANES = 16
DV = D // LANES


def _add_chunk(tok_v, pos_v):
    @pl.loop(0, R)
    def row_loop(r):
        @pl.loop(0, DV, unroll=16)
        def col_loop(j):
            sl = pl.ds(j * LANES, LANES)
            plsc.addupdate(tok_v.at[r, sl], pos_v[r, sl])


def _body(
    idx_hbm, tok_hbm, pos_hbm, out_hbm,
    idx_v, tok0, tok1, tok2, tok3, tok4, pos0, pos1,
    sg0, sg1, sg2, sg3, sg4, sw0, sw1, sw2, sw3, sw4, sp0, sp1, si,
):
    c = lax.axis_index("c")
    s = lax.axis_index("s")
    wid = s * NC + c
    t0 = wid * T_PER_W

    toks = (tok0, tok1, tok2, tok3, tok4)
    poss = (pos0, pos1)
    sgs = (sg0, sg1, sg2, sg3, sg4)
    sws = (sw0, sw1, sw2, sw3, sw4)
    sps = (sp0, sp1)

    for b in range(B):
        pltpu.async_copy(
            idx_hbm.at[pl.ds(b * T + t0, T_PER_W)],
            idx_v.at[pl.ds(b * T_PER_W, T_PER_W)],
            si,
        )
    for b in range(B):
        pltpu.make_async_copy(
            idx_hbm.at[pl.ds(0, T_PER_W)],
            idx_v.at[pl.ds(b * T_PER_W, T_PER_W)],
            si,
        ).wait()

    def fetch_tok(k):
        ci, b = k // B, k % B
        slot = k % NSLOT
        off = b * T_PER_W + ci * R
        pltpu.async_copy(tok_hbm.at[idx_v.at[pl.ds(off, R)]], toks[slot], sgs[slot])

    def wait_tok(k):
        slot = k % NSLOT
        pltpu.make_async_copy(tok_hbm.at[pl.ds(0, R)], toks[slot], sgs[slot]).wait()

    def start_write(k):
        ci, b = k // B, k % B
        slot = k % NSLOT
        pltpu.async_copy(
            toks[slot], out_hbm.at[pl.ds(b * T + t0 + ci * R, R)], sws[slot]
        )

    def wait_write(k):
        slot = k % NSLOT
        pltpu.make_async_copy(toks[slot], out_hbm.at[pl.ds(0, R)], sws[slot]).wait()

    def fetch_pos(ci):
        pltpu.async_copy(
            pos_hbm.at[pl.ds(t0 + ci * R, R)], poss[ci % 2], sps[ci % 2]
        )

    def wait_pos(ci):
        pltpu.make_async_copy(
            pos_hbm.at[pl.ds(0, R)], poss[ci % 2], sps[ci % 2]
        ).wait()

    fetch_pos(0)
    fetch_tok(0)
    fetch_tok(1)
    fetch_tok(2)

    for k in range(NSTEP):
        ci, b = k // B, k % B
        if k + 3 < NSTEP:
            if k >= 2:
                wait_write(k - 2)
            fetch_tok(k + 3)
        if b == 0:
            wait_pos(ci)
            if ci + 1 < NCHUNK:
                fetch_pos(ci + 1)
        wait_tok(k)
        _add_chunk(toks[k % NSLOT], poss[ci % 2])
        start_write(k)

    for k in range(NSTEP - 5, NSTEP):
        wait_write(k)


@jax.jit
def _run(idx_flat, tok_table, pos_table):
    mesh = plsc.VectorSubcoreMesh(
        core_axis_name="c", subcore_axis_name="s", num_cores=NC, num_subcores=NS
    )
    f = pl.kernel(
        _body,
        out_type=jax.ShapeDtypeStruct((N, D), jnp.float32),
        mesh=mesh,
        scratch_types=[
            pltpu.VMEM((B * T_PER_W,), jnp.int32),
            pltpu.VMEM((R, D), jnp.float32),
            pltpu.VMEM((R, D), jnp.float32),
            pltpu.VMEM((R, D), jnp.float32),
            pltpu.VMEM((R, D), jnp.float32),
            pltpu.VMEM((R, D), jnp.float32),
            pltpu.VMEM((R, D), jnp.float32),
            pltpu.VMEM((R, D), jnp.float32),
            pltpu.SemaphoreType.DMA,
            pltpu.SemaphoreType.DMA,
            pltpu.SemaphoreType.DMA,
            pltpu.SemaphoreType.DMA,
            pltpu.SemaphoreType.DMA,
            pltpu.SemaphoreType.DMA,
            pltpu.SemaphoreType.DMA,
            pltpu.SemaphoreType.DMA,
            pltpu.SemaphoreType.DMA,
            pltpu.SemaphoreType.DMA,
            pltpu.SemaphoreType.DMA,
            pltpu.SemaphoreType.DMA,
            pltpu.SemaphoreType.DMA,
        ],
    )
    return f(idx_flat, tok_table, pos_table)


def kernel(idx, tok_table, pos_table):
    idx_flat = idx.reshape(N).astype(jnp.int32)
    out = _run(idx_flat, tok_table, pos_table)
    return out.reshape(B, T, D)
